# initial kernel scaffold (unmeasured)
import jax
import jax.numpy as jnp
from jax import lax
from jax.experimental import pallas as pl
from jax.experimental.pallas import tpu as pltpu

N_DEV = 16
SQ = 256
D_MODEL = 1024
SKV = 4096
H_PER = 8
DH = 128
ROWS = SQ // N_DEV
SCALE = 0.08838834764831843


def _attn_body(x_ref, wq_ref, wo_ref, k_hbm, v_hbm, out_ref,
               k_vmem, v_vmem, o_vmem, copy_sems):
    me = lax.axis_index("i")
    head0 = me * H_PER

    kcp = pltpu.make_async_copy(
        k_hbm.at[0, :, pl.ds(head0, H_PER), :], k_vmem, copy_sems.at[0])
    vcp = pltpu.make_async_copy(
        v_hbm.at[0, :, pl.ds(head0, H_PER), :], v_vmem, copy_sems.at[1])
    kcp.start()
    vcp.start()

    q = jnp.dot(x_ref[0], wq_ref[...],
                preferred_element_type=jnp.float32)

    kcp.wait()
    vcp.wait()

    for h in range(H_PER):
        qh = q[:, h * DH:(h + 1) * DH]
        kh = k_vmem[:, h, :]
        vh = v_vmem[:, h, :]
        s = lax.dot_general(
            qh, kh, (((1,), (1,)), ((), ())),
            preferred_element_type=jnp.float32) * SCALE
        m = jnp.max(s, axis=1, keepdims=True)
        p = jnp.exp(s - m)
        l = jnp.sum(p, axis=1, keepdims=True)
        oh = jnp.dot(p, vh, preferred_element_type=jnp.float32) / l
        o_vmem[:, h * DH:(h + 1) * DH] = oh

    out_ref[...] = jnp.dot(o_vmem[...], wo_ref[...],
                           preferred_element_type=jnp.float32)


def _allreduce_body(part_ref, out_ref, p1_buf, p1_sems, p2_sems, send_sems):
    me = lax.axis_index("i")

    p1_descs = []
    for k in range(1, N_DEV):
        peer = lax.rem(me + k, N_DEV)
        d = pltpu.make_async_remote_copy(
            src_ref=part_ref.at[pl.ds(peer * ROWS, ROWS), :],
            dst_ref=p1_buf.at[N_DEV - k],
            send_sem=send_sems.at[0, k],
            recv_sem=p1_sems.at[N_DEV - k],
            device_id=(peer,),
            device_id_type=pl.DeviceIdType.MESH,
        )
        d.start()
        p1_descs.append(d)

    acc = part_ref[pl.ds(me * ROWS, ROWS), :]
    for k in range(1, N_DEV):
        r = pltpu.make_async_remote_copy(
            src_ref=part_ref.at[pl.ds(0, ROWS), :],
            dst_ref=p1_buf.at[k],
            send_sem=send_sems.at[0, 0],
            recv_sem=p1_sems.at[k],
            device_id=(me,),
            device_id_type=pl.DeviceIdType.MESH,
        )
        r.wait_recv()
        acc = acc + p1_buf[k]
    out_ref[0, pl.ds(me * ROWS, ROWS), :] = acc

    p2_descs = []
    for k in range(1, N_DEV):
        peer = lax.rem(me + k, N_DEV)
        d = pltpu.make_async_remote_copy(
            src_ref=out_ref.at[0, pl.ds(me * ROWS, ROWS), :],
            dst_ref=out_ref.at[0, pl.ds(me * ROWS, ROWS), :],
            send_sem=send_sems.at[1, k],
            recv_sem=p2_sems.at[N_DEV - k],
            device_id=(peer,),
            device_id_type=pl.DeviceIdType.MESH,
        )
        d.start()
        p2_descs.append(d)

    for k in range(1, N_DEV):
        src_chunk = lax.rem(me + N_DEV - k, N_DEV)
        r = pltpu.make_async_remote_copy(
            src_ref=out_ref.at[0, pl.ds(0, ROWS), :],
            dst_ref=out_ref.at[0, pl.ds(src_chunk * ROWS, ROWS), :],
            send_sem=send_sems.at[1, 0],
            recv_sem=p2_sems.at[k],
            device_id=(me,),
            device_id_type=pl.DeviceIdType.MESH,
        )
        r.wait_recv()

    for d in p1_descs + p2_descs:
        d.wait_send()


def kernel(x, Wq, Wo, K_ext, V_ext):
    partial = pl.pallas_call(
        _attn_body,
        out_shape=jax.ShapeDtypeStruct((SQ, D_MODEL), jnp.float32),
        in_specs=[
            pl.BlockSpec(memory_space=pltpu.VMEM),
            pl.BlockSpec(memory_space=pltpu.VMEM),
            pl.BlockSpec(memory_space=pltpu.VMEM),
            pl.BlockSpec(memory_space=pltpu.ANY),
            pl.BlockSpec(memory_space=pltpu.ANY),
        ],
        out_specs=pl.BlockSpec(memory_space=pltpu.VMEM),
        scratch_shapes=[
            pltpu.VMEM((SKV, H_PER, DH), jnp.float32),
            pltpu.VMEM((SKV, H_PER, DH), jnp.float32),
            pltpu.VMEM((SQ, D_MODEL), jnp.float32),
            pltpu.SemaphoreType.DMA((2,)),
        ],
    )(x, Wq, Wo, K_ext, V_ext)

    out = pl.pallas_call(
        _allreduce_body,
        out_shape=jax.ShapeDtypeStruct((1, SQ, D_MODEL), jnp.float32),
        in_specs=[pl.BlockSpec(memory_space=pltpu.VMEM)],
        out_specs=pl.BlockSpec(memory_space=pltpu.VMEM),
        scratch_shapes=[
            pltpu.VMEM((N_DEV, ROWS, D_MODEL), jnp.float32),
            pltpu.SemaphoreType.DMA((N_DEV,)),
            pltpu.SemaphoreType.DMA((N_DEV,)),
            pltpu.SemaphoreType.DMA((2, N_DEV)),
        ],
        compiler_params=pltpu.CompilerParams(collective_id=0),
    )(partial)
    return out


# baseline (device time: 66935 ns/iter reference)
import jax
import jax.numpy as jnp
from jax import lax
from jax.experimental import pallas as pl
from jax.experimental.pallas import tpu as pltpu

N_DEV = 16
SQ = 256
D_MODEL = 1024
SKV = 4096
H_PER = 8
DH = 128
ROWS = SQ // N_DEV
SCALE = 0.08838834764831843


def _attn_body(x_ref, wq_ref, wo_ref, k_hbm, v_hbm, out_ref,
               k_buf, v_buf, o_vmem, copy_sems):
    me = lax.axis_index("i")
    head0 = me * H_PER

    def head_copies(h, slot):
        kcp = pltpu.make_async_copy(
            k_hbm.at[0, :, head0 + h, :], k_buf.at[slot],
            copy_sems.at[0, slot])
        vcp = pltpu.make_async_copy(
            v_hbm.at[0, :, head0 + h, :], v_buf.at[slot],
            copy_sems.at[1, slot])
        return kcp, vcp

    kcp, vcp = head_copies(0, 0)
    kcp.start()
    vcp.start()

    q = jnp.dot(x_ref[0], wq_ref[...],
                preferred_element_type=jnp.float32)

    for h in range(H_PER):
        slot = h % 2
        kcp, vcp = head_copies(h, slot)
        kcp.wait()
        vcp.wait()
        if h + 1 < H_PER:
            nkcp, nvcp = head_copies(h + 1, (h + 1) % 2)
            nkcp.start()
            nvcp.start()
        qh = q[:, h * DH:(h + 1) * DH]
        kh = k_buf[slot]
        vh = v_buf[slot]
        s = lax.dot_general(
            qh, kh, (((1,), (1,)), ((), ())),
            preferred_element_type=jnp.float32) * SCALE
        m = jnp.max(s, axis=1, keepdims=True)
        p = jnp.exp(s - m)
        l = jnp.sum(p, axis=1, keepdims=True)
        oh = jnp.dot(p, vh, preferred_element_type=jnp.float32) / l
        o_vmem[:, h * DH:(h + 1) * DH] = oh

    out_ref[...] = jnp.dot(o_vmem[...], wo_ref[...],
                           preferred_element_type=jnp.float32)


def _allreduce_body(part_ref, out_ref, p1_buf, p1_sems, p2_sems, send_sems):
    me = lax.axis_index("i")

    p1_descs = []
    for k in range(1, N_DEV):
        peer = lax.rem(me + k, N_DEV)
        d = pltpu.make_async_remote_copy(
            src_ref=part_ref.at[pl.ds(peer * ROWS, ROWS), :],
            dst_ref=p1_buf.at[N_DEV - k],
            send_sem=send_sems.at[0, k],
            recv_sem=p1_sems.at[N_DEV - k],
            device_id=(peer,),
            device_id_type=pl.DeviceIdType.MESH,
        )
        d.start()
        p1_descs.append(d)

    acc = part_ref[pl.ds(me * ROWS, ROWS), :]
    for k in range(1, N_DEV):
        r = pltpu.make_async_remote_copy(
            src_ref=part_ref.at[pl.ds(0, ROWS), :],
            dst_ref=p1_buf.at[k],
            send_sem=send_sems.at[0, 0],
            recv_sem=p1_sems.at[k],
            device_id=(me,),
            device_id_type=pl.DeviceIdType.MESH,
        )
        r.wait_recv()
        acc = acc + p1_buf[k]
    out_ref[0, pl.ds(me * ROWS, ROWS), :] = acc

    p2_descs = []
    for k in range(1, N_DEV):
        peer = lax.rem(me + k, N_DEV)
        d = pltpu.make_async_remote_copy(
            src_ref=out_ref.at[0, pl.ds(me * ROWS, ROWS), :],
            dst_ref=out_ref.at[0, pl.ds(me * ROWS, ROWS), :],
            send_sem=send_sems.at[1, k],
            recv_sem=p2_sems.at[N_DEV - k],
            device_id=(peer,),
            device_id_type=pl.DeviceIdType.MESH,
        )
        d.start()
        p2_descs.append(d)

    for k in range(1, N_DEV):
        src_chunk = lax.rem(me + k, N_DEV)
        r = pltpu.make_async_remote_copy(
            src_ref=out_ref.at[0, pl.ds(0, ROWS), :],
            dst_ref=out_ref.at[0, pl.ds(src_chunk * ROWS, ROWS), :],
            send_sem=send_sems.at[1, 0],
            recv_sem=p2_sems.at[k],
            device_id=(me,),
            device_id_type=pl.DeviceIdType.MESH,
        )
        r.wait_recv()

    for d in p1_descs + p2_descs:
        d.wait_send()


def kernel(x, Wq, Wo, K_ext, V_ext):
    partial = pl.pallas_call(
        _attn_body,
        out_shape=jax.ShapeDtypeStruct((SQ, D_MODEL), jnp.float32),
        in_specs=[
            pl.BlockSpec(memory_space=pltpu.VMEM),
            pl.BlockSpec(memory_space=pltpu.VMEM),
            pl.BlockSpec(memory_space=pltpu.VMEM),
            pl.BlockSpec(memory_space=pltpu.MemorySpace.HBM),
            pl.BlockSpec(memory_space=pltpu.MemorySpace.HBM),
        ],
        out_specs=pl.BlockSpec(memory_space=pltpu.VMEM),
        scratch_shapes=[
            pltpu.VMEM((2, SKV, DH), jnp.float32),
            pltpu.VMEM((2, SKV, DH), jnp.float32),
            pltpu.VMEM((SQ, D_MODEL), jnp.float32),
            pltpu.SemaphoreType.DMA((2, 2)),
        ],
    )(x, Wq, Wo, K_ext, V_ext)

    out = pl.pallas_call(
        _allreduce_body,
        out_shape=jax.ShapeDtypeStruct((1, SQ, D_MODEL), jnp.float32),
        in_specs=[pl.BlockSpec(memory_space=pltpu.VMEM)],
        out_specs=pl.BlockSpec(memory_space=pltpu.VMEM),
        scratch_shapes=[
            pltpu.VMEM((N_DEV, ROWS, D_MODEL), jnp.float32),
            pltpu.SemaphoreType.DMA((N_DEV,)),
            pltpu.SemaphoreType.DMA((N_DEV,)),
            pltpu.SemaphoreType.DMA((2, N_DEV)),
        ],
    )(partial)
    return out


# device time: 61301 ns/iter; 1.0919x vs baseline; 1.0919x over previous
import jax
import jax.numpy as jnp
from jax import lax
from jax.experimental import pallas as pl
from jax.experimental.pallas import tpu as pltpu

N_DEV = 16
SQ = 256
D_MODEL = 1024
SKV = 4096
H_PER = 8
DH = 128
ROWS = SQ // N_DEV
SCALE = 0.08838834764831843


def _bf(x):
    return x.astype(jnp.bfloat16)


def _fused_body(x_ref, wq_ref, wo_ref, k_hbm, v_hbm, out_ref,
                k_buf, v_buf, part_ref, p1_buf,
                copy_sems, p1_sems, p2_sems, send_sems):
    me = lax.axis_index("i")
    head0 = me * H_PER

    def head_copies(h, slot):
        kcp = pltpu.make_async_copy(
            k_hbm.at[0, :, head0 + h, :], k_buf.at[slot],
            copy_sems.at[0, slot])
        vcp = pltpu.make_async_copy(
            v_hbm.at[0, :, head0 + h, :], v_buf.at[slot],
            copy_sems.at[1, slot])
        return kcp, vcp

    kcp, vcp = head_copies(0, 0)
    kcp.start()
    vcp.start()

    q = jnp.dot(_bf(x_ref[0]), _bf(wq_ref[...]),
                preferred_element_type=jnp.float32)

    partial = jnp.zeros((SQ, D_MODEL), jnp.float32)
    for h in range(H_PER):
        slot = h % 2
        kcp, vcp = head_copies(h, slot)
        kcp.wait()
        vcp.wait()
        if h + 1 < H_PER:
            nkcp, nvcp = head_copies(h + 1, (h + 1) % 2)
            nkcp.start()
            nvcp.start()
        qh = _bf(q[:, h * DH:(h + 1) * DH])
        kh = _bf(k_buf[slot])
        vh = _bf(v_buf[slot])
        s = lax.dot_general(
            qh, kh, (((1,), (1,)), ((), ())),
            preferred_element_type=jnp.float32) * SCALE
        m = jnp.max(s, axis=1, keepdims=True)
        p = jnp.exp(s - m)
        l = jnp.sum(p, axis=1, keepdims=True)
        oh = jnp.dot(_bf(p), vh, preferred_element_type=jnp.float32) / l
        partial = partial + jnp.dot(
            _bf(oh), _bf(wo_ref[pl.ds(h * DH, DH), :]),
            preferred_element_type=jnp.float32)
    part_ref[...] = partial

    p1_descs = []
    for k in range(1, N_DEV):
        peer = lax.rem(me + k, N_DEV)
        d = pltpu.make_async_remote_copy(
            src_ref=part_ref.at[pl.ds(peer * ROWS, ROWS), :],
            dst_ref=p1_buf.at[N_DEV - k],
            send_sem=send_sems.at[0, k],
            recv_sem=p1_sems.at[N_DEV - k],
            device_id=(peer,),
            device_id_type=pl.DeviceIdType.MESH,
        )
        d.start()
        p1_descs.append(d)

    acc = part_ref[pl.ds(me * ROWS, ROWS), :]
    for k in range(1, N_DEV):
        r = pltpu.make_async_remote_copy(
            src_ref=part_ref.at[pl.ds(0, ROWS), :],
            dst_ref=p1_buf.at[k],
            send_sem=send_sems.at[0, 0],
            recv_sem=p1_sems.at[k],
            device_id=(me,),
            device_id_type=pl.DeviceIdType.MESH,
        )
        r.wait_recv()
        acc = acc + p1_buf[k]
    out_ref[0, pl.ds(me * ROWS, ROWS), :] = acc

    p2_descs = []
    for k in range(1, N_DEV):
        peer = lax.rem(me + k, N_DEV)
        d = pltpu.make_async_remote_copy(
            src_ref=out_ref.at[0, pl.ds(me * ROWS, ROWS), :],
            dst_ref=out_ref.at[0, pl.ds(me * ROWS, ROWS), :],
            send_sem=send_sems.at[1, k],
            recv_sem=p2_sems.at[N_DEV - k],
            device_id=(peer,),
            device_id_type=pl.DeviceIdType.MESH,
        )
        d.start()
        p2_descs.append(d)

    for k in range(1, N_DEV):
        src_chunk = lax.rem(me + k, N_DEV)
        r = pltpu.make_async_remote_copy(
            src_ref=out_ref.at[0, pl.ds(0, ROWS), :],
            dst_ref=out_ref.at[0, pl.ds(src_chunk * ROWS, ROWS), :],
            send_sem=send_sems.at[1, 0],
            recv_sem=p2_sems.at[k],
            device_id=(me,),
            device_id_type=pl.DeviceIdType.MESH,
        )
        r.wait_recv()

    for d in p1_descs + p2_descs:
        d.wait_send()


def kernel(x, Wq, Wo, K_ext, V_ext):
    return pl.pallas_call(
        _fused_body,
        out_shape=jax.ShapeDtypeStruct((1, SQ, D_MODEL), jnp.float32),
        in_specs=[
            pl.BlockSpec(memory_space=pltpu.VMEM),
            pl.BlockSpec(memory_space=pltpu.VMEM),
            pl.BlockSpec(memory_space=pltpu.VMEM),
            pl.BlockSpec(memory_space=pltpu.MemorySpace.HBM),
            pl.BlockSpec(memory_space=pltpu.MemorySpace.HBM),
        ],
        out_specs=pl.BlockSpec(memory_space=pltpu.VMEM),
        scratch_shapes=[
            pltpu.VMEM((2, SKV, DH), jnp.float32),
            pltpu.VMEM((2, SKV, DH), jnp.float32),
            pltpu.VMEM((SQ, D_MODEL), jnp.float32),
            pltpu.VMEM((N_DEV, ROWS, D_MODEL), jnp.float32),
            pltpu.SemaphoreType.DMA((2, 2)),
            pltpu.SemaphoreType.DMA((N_DEV,)),
            pltpu.SemaphoreType.DMA((N_DEV,)),
            pltpu.SemaphoreType.DMA((2, N_DEV)),
        ],
    )(x, Wq, Wo, K_ext, V_ext)
